# Initial kernel scaffold; baseline (speedup 1.0000x reference)
import jax
import jax.numpy as jnp
from jax.experimental import pallas as pl


def kernel(x, x_demo, sorted_length, W_ih, W_hh, b_ih, b_hh, h0, Wq, bq, Wk, bk, Wo_w, Wo_b, phi, Wg, bg, W_pre, b_pre):
    raise NotImplementedError("write your pallas kernel here")


# speedup vs baseline: 2.0993x; 2.0993x over previous
import jax
import jax.numpy as jnp
from jax.experimental import pallas as pl
from jax.experimental.pallas import tpu as pltpu

B = 2048
T = 20
D_IN = 128
H = 128
D_DEMO = 16
DZ = H + D_DEMO
HEADS = 4
D_K = DZ // HEADS
RB = 256
NRB = B // RB
YP = 8


def _gru_body(x2d_ref, xdemo_ref, idx_ref, wihT_ref, whhT_ref, bih_ref,
              bhh_ref, h0_ref, z_ref, h_s, last_s):
    t = pl.program_id(0)

    @pl.when(t == 0)
    def _():
        h_s[...] = jnp.broadcast_to(h0_ref[...], (B, H))

    x_t = x2d_ref[...]
    h = h_s[...]
    gi = jnp.dot(x_t, wihT_ref[...], preferred_element_type=jnp.float32) + bih_ref[...]
    gh = jnp.dot(h, whhT_ref[...], preferred_element_type=jnp.float32) + bhh_ref[...]
    r = jax.nn.sigmoid(gi[:, 0:H] + gh[:, 0:H])
    zg = jax.nn.sigmoid(gi[:, H:2 * H] + gh[:, H:2 * H])
    n = jnp.tanh(gi[:, 2 * H:3 * H] + r * gh[:, 2 * H:3 * H])
    h_new = (1.0 - zg) * n + zg * h
    h_s[...] = h_new
    sel = idx_ref[...] == t
    last_s[...] = jnp.where(sel, h_new, last_s[...])

    @pl.when(t == T - 1)
    def _():
        z_ref[:, 0:H] = last_s[...]
        z_ref[:, H:DZ] = xdemo_ref[...]


def _proj_body(z_ref, wqT_ref, bq_ref, wkT_ref, bk_ref, wg2T_ref,
               q_ref, k_ref, y2_ref):
    z = z_ref[...]
    q_ref[...] = jnp.dot(z, wqT_ref[...], preferred_element_type=jnp.float32) + bq_ref[...]
    k_ref[...] = jnp.dot(z, wkT_ref[...], preferred_element_type=jnp.float32) + bk_ref[...]
    y2_ref[...] = jnp.dot(z, wg2T_ref[...], preferred_element_type=jnp.float32)


def _scores(q_blk, k_all, wob_ref):
    s = jax.lax.dot_general(q_blk, k_all, (((1,), (1,)), ((), ())),
                            preferred_element_type=jnp.float32)
    return s + wob_ref[...]


def _stats_body(q_ref, k_ref, wob_ref, phi_ref, stats_ref):
    s = _scores(q_ref[...], k_ref[...], wob_ref)
    m = jnp.max(s, axis=1, keepdims=True)
    e = jnp.exp(s - m)
    den = jnp.sum(e, axis=1, keepdims=True)
    maskf = (e >= phi_ref[...] * den).astype(jnp.float32)
    deg = jnp.sum(maskf, axis=1, keepdims=True) + 1.0
    dinv = jax.lax.rsqrt(deg)
    stats_ref[...] = jnp.zeros((RB, YP), jnp.float32)
    stats_ref[:, 0:1] = m
    stats_ref[:, 1:2] = den
    stats_ref[:, 2:3] = dinv


def _agg_body(q_ref, k_ref, stats_ref, y2_ref, wob_ref, phi_ref, b2_ref,
              out_ref):
    i = pl.program_id(0)
    s = _scores(q_ref[...], k_ref[...], wob_ref)
    st = stats_ref[pl.ds(i * RB, RB), :]
    m = st[:, 0:1]
    den = st[:, 1:2]
    dinv_i = st[:, 2:3]
    e = jnp.exp(s - m)
    maskf = (e >= phi_ref[...] * den).astype(jnp.float32)
    dinv_all = stats_ref[:, 2:3]
    yd = y2_ref[...] * dinv_all
    acc = jnp.dot(maskf, yd, preferred_element_type=jnp.float32)
    y2_i = y2_ref[pl.ds(i * RB, RB), :]
    out_ref[...] = dinv_i * acc + (dinv_i * dinv_i) * y2_i + b2_ref[...]


def kernel(x, x_demo, sorted_length, W_ih, W_hh, b_ih, b_hh, h0, Wq, bq,
           Wk, bk, Wo_w, Wo_b, phi, Wg, bg, W_pre, b_pre):
    f32 = jnp.float32
    x2d = x.reshape(B, T * D_IN)
    idx = jnp.clip(sorted_length.astype(jnp.int32) - 1, 0, T - 1).reshape(B, 1)

    scale = (Wo_w[0] / jnp.sqrt(f32(D_K))).repeat(D_K)
    wqT = (Wq * scale[:, None]).T
    bq_eff = (bq * scale).reshape(1, DZ)
    wkT = Wk.T
    bk2 = bk.reshape(1, DZ)
    wg2 = W_pre @ Wg
    wg2T = jnp.zeros((DZ, YP), f32).at[:, 0:2].set(wg2.T)
    b2 = jnp.zeros((1, YP), f32).at[0, 0:2].set(W_pre @ bg + b_pre)
    wob = Wo_b.reshape(1, 1)
    phi2 = jnp.asarray(phi, f32).reshape(1, 1)

    z = pl.pallas_call(
        _gru_body,
        grid=(T,),
        in_specs=[
            pl.BlockSpec((B, D_IN), lambda t: (0, t)),
            pl.BlockSpec((B, D_DEMO), lambda t: (0, 0)),
            pl.BlockSpec((B, 1), lambda t: (0, 0)),
            pl.BlockSpec((D_IN, 3 * H), lambda t: (0, 0)),
            pl.BlockSpec((H, 3 * H), lambda t: (0, 0)),
            pl.BlockSpec((1, 3 * H), lambda t: (0, 0)),
            pl.BlockSpec((1, 3 * H), lambda t: (0, 0)),
            pl.BlockSpec((1, H), lambda t: (0, 0)),
        ],
        out_specs=pl.BlockSpec((B, DZ), lambda t: (0, 0)),
        out_shape=jax.ShapeDtypeStruct((B, DZ), f32),
        scratch_shapes=[pltpu.VMEM((B, H), f32), pltpu.VMEM((B, H), f32)],
        compiler_params=pltpu.CompilerParams(
            dimension_semantics=("arbitrary",)),
    )(x2d, x_demo, idx, W_ih.T, W_hh.T, b_ih.reshape(1, 3 * H),
      b_hh.reshape(1, 3 * H), h0.reshape(1, H))

    q, k, y2 = pl.pallas_call(
        _proj_body,
        out_shape=(
            jax.ShapeDtypeStruct((B, DZ), f32),
            jax.ShapeDtypeStruct((B, DZ), f32),
            jax.ShapeDtypeStruct((B, YP), f32),
        ),
    )(z, wqT, bq_eff, wkT, bk2, wg2T)

    stats = pl.pallas_call(
        _stats_body,
        grid=(NRB,),
        in_specs=[
            pl.BlockSpec((RB, DZ), lambda i: (i, 0)),
            pl.BlockSpec((B, DZ), lambda i: (0, 0)),
            pl.BlockSpec((1, 1), lambda i: (0, 0)),
            pl.BlockSpec((1, 1), lambda i: (0, 0)),
        ],
        out_specs=pl.BlockSpec((RB, YP), lambda i: (i, 0)),
        out_shape=jax.ShapeDtypeStruct((B, YP), f32),
        compiler_params=pltpu.CompilerParams(
            dimension_semantics=("arbitrary",)),
    )(q, k, wob, phi2)

    logits_pad = pl.pallas_call(
        _agg_body,
        grid=(NRB,),
        in_specs=[
            pl.BlockSpec((RB, DZ), lambda i: (i, 0)),
            pl.BlockSpec((B, DZ), lambda i: (0, 0)),
            pl.BlockSpec((B, YP), lambda i: (0, 0)),
            pl.BlockSpec((B, YP), lambda i: (0, 0)),
            pl.BlockSpec((1, 1), lambda i: (0, 0)),
            pl.BlockSpec((1, 1), lambda i: (0, 0)),
            pl.BlockSpec((1, YP), lambda i: (0, 0)),
        ],
        out_specs=pl.BlockSpec((RB, YP), lambda i: (i, 0)),
        out_shape=jax.ShapeDtypeStruct((B, YP), f32),
        compiler_params=pltpu.CompilerParams(
            dimension_semantics=("arbitrary",)),
    )(q, k, stats, y2, wob, phi2, b2)

    return logits_pad[:, 0:2]


# speedup vs baseline: 2.3954x; 1.1410x over previous
import jax
import jax.numpy as jnp
from jax.experimental import pallas as pl
from jax.experimental.pallas import tpu as pltpu

B = 2048
T = 20
D_IN = 128
H = 128
D_DEMO = 16
DZ = H + D_DEMO
HEADS = 4
D_K = DZ // HEADS
RB = 256
NRB = B // RB
YP = 8


def _gru_body(x2d_ref, xdemo_ref, idx_ref, wcat_ref, bcat_ref, h0_ref,
              z_ref, h_s, xh_s):
    t = pl.program_id(0)

    @pl.when(t == 0)
    def _():
        h_s[...] = jnp.broadcast_to(h0_ref[...], (B, H))

    h = h_s[...]
    xh_s[:, 0:D_IN] = x2d_ref[...]
    xh_s[:, D_IN:D_IN + H] = h
    g = jnp.dot(xh_s[...], wcat_ref[...],
                preferred_element_type=jnp.float32) + bcat_ref[...]
    rz = jax.nn.sigmoid(g[:, 0:2 * H])
    r = rz[:, 0:H]
    zg = rz[:, H:2 * H]
    n = jnp.tanh(g[:, 2 * H:3 * H] + r * g[:, 3 * H:4 * H])
    h_new = (1.0 - zg) * n + zg * h
    h_s[...] = h_new
    sel = idx_ref[...] == t
    z_ref[:, 0:H] = jnp.where(sel, h_new, z_ref[:, 0:H])

    @pl.when(t == T - 1)
    def _():
        z_ref[:, H:DZ] = xdemo_ref[...]


def _graph_body(z_ref, wqT_ref, bq_ref, wkT_ref, bk_ref, wg2p_ref, wob_ref,
                phi_ref, b2_ref, eye_ref, out_ref,
                s_s, q_s, k_s, y2t_s, ydt_s, thr_s, dinvrow_s):
    p = pl.program_id(0)
    i = pl.program_id(1)

    @pl.when(jnp.logical_and(p == 0, i == 0))
    def _():
        z = z_ref[...]
        q_s[...] = jnp.dot(z, wqT_ref[...], preferred_element_type=jnp.float32) + bq_ref[...]
        k_s[...] = jnp.dot(z, wkT_ref[...], preferred_element_type=jnp.float32) + bk_ref[...]
        y2t_s[...] = jax.lax.dot_general(
            wg2p_ref[...], z, (((1,), (1,)), ((), ())),
            preferred_element_type=jnp.float32)

    @pl.when(p == 0)
    def _():
        q_i = q_s[pl.ds(i * RB, RB), :]
        s = jax.lax.dot_general(q_i, k_s[...], (((1,), (1,)), ((), ())),
                                preferred_element_type=jnp.float32) + wob_ref[...]
        s_s[pl.ds(i * RB, RB), :] = s
        m = jnp.max(s, axis=1, keepdims=True)
        e = jnp.exp(s - m)
        den = jnp.sum(e, axis=1, keepdims=True)
        thr = m + jnp.log(phi_ref[...] * den)
        deg = jnp.sum((s >= thr).astype(jnp.float32), axis=1,
                      keepdims=True) + 1.0
        dinv = jax.lax.rsqrt(deg)
        thr_s[pl.ds(i * RB, RB), :] = jnp.broadcast_to(thr, (RB, YP))
        dinvrow_s[:, pl.ds(i * RB, RB)] = jax.lax.dot_general(
            dinv, eye_ref[...], (((0,), (0,)), ((), ())),
            preferred_element_type=jnp.float32)

    @pl.when(p == 1)
    def _():
        @pl.when(i == 0)
        def _():
            ydt_s[...] = y2t_s[...] * dinvrow_s[...]

        s = s_s[pl.ds(i * RB, RB), :]
        thr = thr_s[pl.ds(i * RB, RB), 0:1]
        maskf = (s >= thr).astype(jnp.float32)
        accT = jax.lax.dot_general(
            ydt_s[...], maskf, (((1,), (1,)), ((), ())),
            preferred_element_type=jnp.float32)
        dr_i = dinvrow_s[:, pl.ds(i * RB, RB)]
        y2t_i = y2t_s[:, pl.ds(i * RB, RB)]
        out_ref[...] = dr_i * accT + (dr_i * dr_i) * y2t_i + b2_ref[...]


def kernel(x, x_demo, sorted_length, W_ih, W_hh, b_ih, b_hh, h0, Wq, bq,
           Wk, bk, Wo_w, Wo_b, phi, Wg, bg, W_pre, b_pre):
    f32 = jnp.float32
    x2d = x.reshape(B, T * D_IN)
    idx = jnp.clip(sorted_length.astype(jnp.int32) - 1, 0, T - 1).reshape(B, 1)

    wihT = W_ih.T
    whhT = W_hh.T
    wcat = jnp.zeros((D_IN + H, 4 * H), f32)
    wcat = wcat.at[0:D_IN, 0:2 * H].set(wihT[:, 0:2 * H])
    wcat = wcat.at[D_IN:, 0:2 * H].set(whhT[:, 0:2 * H])
    wcat = wcat.at[0:D_IN, 2 * H:3 * H].set(wihT[:, 2 * H:3 * H])
    wcat = wcat.at[D_IN:, 3 * H:4 * H].set(whhT[:, 2 * H:3 * H])
    bcat = jnp.concatenate([
        (b_ih[0:2 * H] + b_hh[0:2 * H]),
        b_ih[2 * H:3 * H],
        b_hh[2 * H:3 * H],
    ]).reshape(1, 4 * H)

    scale = (Wo_w[0] / jnp.sqrt(f32(D_K))).repeat(D_K)
    wqT = (Wq * scale[:, None]).T
    bq_eff = (bq * scale).reshape(1, DZ)
    wkT = Wk.T
    bk2 = bk.reshape(1, DZ)
    wg2p = jnp.zeros((YP, DZ), f32).at[0:2, :].set(W_pre @ Wg)
    b2 = jnp.zeros((YP, 1), f32).at[0:2, 0].set(W_pre @ bg + b_pre)
    wob = Wo_b.reshape(1, 1)
    phi2 = jnp.asarray(phi, f32).reshape(1, 1)
    eye = jnp.eye(RB, dtype=f32)

    z = pl.pallas_call(
        _gru_body,
        grid=(T,),
        in_specs=[
            pl.BlockSpec((B, D_IN), lambda t: (0, t)),
            pl.BlockSpec((B, D_DEMO), lambda t: (0, 0)),
            pl.BlockSpec((B, 1), lambda t: (0, 0)),
            pl.BlockSpec((D_IN + H, 4 * H), lambda t: (0, 0)),
            pl.BlockSpec((1, 4 * H), lambda t: (0, 0)),
            pl.BlockSpec((1, H), lambda t: (0, 0)),
        ],
        out_specs=pl.BlockSpec((B, DZ), lambda t: (0, 0)),
        out_shape=jax.ShapeDtypeStruct((B, DZ), f32),
        scratch_shapes=[pltpu.VMEM((B, H), f32),
                        pltpu.VMEM((B, D_IN + H), f32)],
        compiler_params=pltpu.CompilerParams(
            dimension_semantics=("arbitrary",)),
    )(x2d, x_demo, idx, wcat, bcat, h0.reshape(1, H))

    outT = pl.pallas_call(
        _graph_body,
        grid=(2, NRB),
        in_specs=[
            pl.BlockSpec((B, DZ), lambda p, i: (0, 0)),
            pl.BlockSpec((DZ, DZ), lambda p, i: (0, 0)),
            pl.BlockSpec((1, DZ), lambda p, i: (0, 0)),
            pl.BlockSpec((DZ, DZ), lambda p, i: (0, 0)),
            pl.BlockSpec((1, DZ), lambda p, i: (0, 0)),
            pl.BlockSpec((YP, DZ), lambda p, i: (0, 0)),
            pl.BlockSpec((1, 1), lambda p, i: (0, 0)),
            pl.BlockSpec((1, 1), lambda p, i: (0, 0)),
            pl.BlockSpec((YP, 1), lambda p, i: (0, 0)),
            pl.BlockSpec((RB, RB), lambda p, i: (0, 0)),
        ],
        out_specs=pl.BlockSpec((YP, RB), lambda p, i: (0, i)),
        out_shape=jax.ShapeDtypeStruct((YP, B), f32),
        scratch_shapes=[
            pltpu.VMEM((B, B), f32),
            pltpu.VMEM((B, DZ), f32),
            pltpu.VMEM((B, DZ), f32),
            pltpu.VMEM((YP, B), f32),
            pltpu.VMEM((YP, B), f32),
            pltpu.VMEM((B, YP), f32),
            pltpu.VMEM((1, B), f32),
        ],
        compiler_params=pltpu.CompilerParams(
            dimension_semantics=("arbitrary", "arbitrary")),
    )(z, wqT, bq_eff, wkT, bk2, wg2p, wob, phi2, b2, eye)

    return outT[0:2, :].T


# speedup vs baseline: 2.6110x; 1.0900x over previous
import jax
import jax.numpy as jnp
from jax.experimental import pallas as pl
from jax.experimental.pallas import tpu as pltpu

B = 2048
T = 20
D_IN = 128
H = 128
D_DEMO = 16
DZ = H + D_DEMO
HEADS = 4
D_K = DZ // HEADS
RB = 256
NRB = B // RB
YP = 8


def _gru_body(x_hbm, xdemo_ref, idx_ref, wcat_ref, bcat_ref, h0_ref,
              z_ref, h_s, xh_s, xbuf, sem):
    t = pl.program_id(0)
    slot = jax.lax.rem(t, 2)
    nslot = jax.lax.rem(t + 1, 2)

    @pl.when(t == 0)
    def _():
        h_s[...] = jnp.broadcast_to(h0_ref[...], (B, H))
        pltpu.make_async_copy(x_hbm.at[:, 0, :], xbuf.at[0], sem.at[0]).start()

    @pl.when(t + 1 < T)
    def _():
        pltpu.make_async_copy(x_hbm.at[:, t + 1, :], xbuf.at[nslot],
                              sem.at[nslot]).start()

    pltpu.make_async_copy(x_hbm.at[:, t, :], xbuf.at[slot],
                          sem.at[slot]).wait()

    h = h_s[...]
    xh_s[:, 0:D_IN] = xbuf[slot]
    xh_s[:, D_IN:D_IN + H] = h
    g = jnp.dot(xh_s[...], wcat_ref[...],
                preferred_element_type=jnp.float32) + bcat_ref[...]
    rz = 0.5 * (jnp.tanh(g[:, 0:2 * H] * 0.5) + 1.0)
    r = rz[:, 0:H]
    zg = rz[:, H:2 * H]
    n = jnp.tanh(g[:, 2 * H:3 * H] + r * g[:, 3 * H:4 * H])
    h_new = (1.0 - zg) * n + zg * h
    h_s[...] = h_new
    sel = idx_ref[...] == t
    z_ref[:, 0:H] = jnp.where(sel, h_new, z_ref[:, 0:H])

    @pl.when(t == T - 1)
    def _():
        z_ref[:, H:DZ] = xdemo_ref[...]


def _graph_body(z_ref, wqT_ref, bq_ref, wkT_ref, bk_ref, wg2p_ref, wob_ref,
                phi_ref, b2_ref, eye_ref, out_ref,
                s_s, q_s, k_s, y2t_s, ydt_s, thr_s, dinvrow_s):
    p = pl.program_id(0)
    i = pl.program_id(1)

    @pl.when(jnp.logical_and(p == 0, i == 0))
    def _():
        z = z_ref[...]
        q_s[...] = jnp.dot(z, wqT_ref[...], preferred_element_type=jnp.float32) + bq_ref[...]
        k_s[...] = jnp.dot(z, wkT_ref[...], preferred_element_type=jnp.float32) + bk_ref[...]
        y2t_s[...] = jax.lax.dot_general(
            wg2p_ref[...], z, (((1,), (1,)), ((), ())),
            preferred_element_type=jnp.float32)

    @pl.when(p == 0)
    def _():
        q_i = q_s[pl.ds(i * RB, RB), :]
        s = jax.lax.dot_general(q_i, k_s[...], (((1,), (1,)), ((), ())),
                                preferred_element_type=jnp.float32) + wob_ref[...]
        s_s[pl.ds(i * RB, RB), :] = s
        m = jnp.max(s, axis=1, keepdims=True)
        e = jnp.exp(s - m)
        den = jnp.sum(e, axis=1, keepdims=True)
        thr = m + jnp.log(phi_ref[...] * den)
        deg = jnp.sum((s >= thr).astype(jnp.float32), axis=1,
                      keepdims=True) + 1.0
        dinv = jax.lax.rsqrt(deg)
        thr_s[pl.ds(i * RB, RB), :] = jnp.broadcast_to(thr, (RB, YP))
        dinvrow_s[:, pl.ds(i * RB, RB)] = jax.lax.dot_general(
            dinv, eye_ref[...], (((0,), (0,)), ((), ())),
            preferred_element_type=jnp.float32)

    @pl.when(p == 1)
    def _():
        @pl.when(i == 0)
        def _():
            ydt_s[...] = y2t_s[...] * dinvrow_s[...]

        s = s_s[pl.ds(i * RB, RB), :]
        thr = thr_s[pl.ds(i * RB, RB), 0:1]
        maskf = (s >= thr).astype(jnp.float32)
        accT = jax.lax.dot_general(
            ydt_s[...], maskf, (((1,), (1,)), ((), ())),
            preferred_element_type=jnp.float32)
        dr_i = dinvrow_s[:, pl.ds(i * RB, RB)]
        y2t_i = y2t_s[:, pl.ds(i * RB, RB)]
        out_ref[...] = dr_i * accT + (dr_i * dr_i) * y2t_i + b2_ref[...]


def kernel(x, x_demo, sorted_length, W_ih, W_hh, b_ih, b_hh, h0, Wq, bq,
           Wk, bk, Wo_w, Wo_b, phi, Wg, bg, W_pre, b_pre):
    f32 = jnp.float32
    idx = jnp.clip(sorted_length.astype(jnp.int32) - 1, 0, T - 1).reshape(B, 1)

    wihT = W_ih.T
    whhT = W_hh.T
    wcat = jnp.zeros((D_IN + H, 4 * H), f32)
    wcat = wcat.at[0:D_IN, 0:2 * H].set(wihT[:, 0:2 * H])
    wcat = wcat.at[D_IN:, 0:2 * H].set(whhT[:, 0:2 * H])
    wcat = wcat.at[0:D_IN, 2 * H:3 * H].set(wihT[:, 2 * H:3 * H])
    wcat = wcat.at[D_IN:, 3 * H:4 * H].set(whhT[:, 2 * H:3 * H])
    bcat = jnp.concatenate([
        (b_ih[0:2 * H] + b_hh[0:2 * H]),
        b_ih[2 * H:3 * H],
        b_hh[2 * H:3 * H],
    ]).reshape(1, 4 * H)

    scale = (Wo_w[0] / jnp.sqrt(f32(D_K))).repeat(D_K)
    wqT = (Wq * scale[:, None]).T
    bq_eff = (bq * scale).reshape(1, DZ)
    wkT = Wk.T
    bk2 = bk.reshape(1, DZ)
    wg2p = jnp.zeros((YP, DZ), f32).at[0:2, :].set(W_pre @ Wg)
    b2 = jnp.zeros((YP, 1), f32).at[0:2, 0].set(W_pre @ bg + b_pre)
    wob = Wo_b.reshape(1, 1)
    phi2 = jnp.asarray(phi, f32).reshape(1, 1)
    eye = jnp.eye(RB, dtype=f32)

    z = pl.pallas_call(
        _gru_body,
        grid=(T,),
        in_specs=[
            pl.BlockSpec(memory_space=pl.ANY),
            pl.BlockSpec((B, D_DEMO), lambda t: (0, 0)),
            pl.BlockSpec((B, 1), lambda t: (0, 0)),
            pl.BlockSpec((D_IN + H, 4 * H), lambda t: (0, 0)),
            pl.BlockSpec((1, 4 * H), lambda t: (0, 0)),
            pl.BlockSpec((1, H), lambda t: (0, 0)),
        ],
        out_specs=pl.BlockSpec((B, DZ), lambda t: (0, 0)),
        out_shape=jax.ShapeDtypeStruct((B, DZ), f32),
        scratch_shapes=[pltpu.VMEM((B, H), f32),
                        pltpu.VMEM((B, D_IN + H), f32),
                        pltpu.VMEM((2, B, D_IN), f32),
                        pltpu.SemaphoreType.DMA((2,))],
        compiler_params=pltpu.CompilerParams(
            dimension_semantics=("arbitrary",)),
    )(x, x_demo, idx, wcat, bcat, h0.reshape(1, H))

    outT = pl.pallas_call(
        _graph_body,
        grid=(2, NRB),
        in_specs=[
            pl.BlockSpec((B, DZ), lambda p, i: (0, 0)),
            pl.BlockSpec((DZ, DZ), lambda p, i: (0, 0)),
            pl.BlockSpec((1, DZ), lambda p, i: (0, 0)),
            pl.BlockSpec((DZ, DZ), lambda p, i: (0, 0)),
            pl.BlockSpec((1, DZ), lambda p, i: (0, 0)),
            pl.BlockSpec((YP, DZ), lambda p, i: (0, 0)),
            pl.BlockSpec((1, 1), lambda p, i: (0, 0)),
            pl.BlockSpec((1, 1), lambda p, i: (0, 0)),
            pl.BlockSpec((YP, 1), lambda p, i: (0, 0)),
            pl.BlockSpec((RB, RB), lambda p, i: (0, 0)),
        ],
        out_specs=pl.BlockSpec((YP, RB), lambda p, i: (0, i)),
        out_shape=jax.ShapeDtypeStruct((YP, B), f32),
        scratch_shapes=[
            pltpu.VMEM((B, B), f32),
            pltpu.VMEM((B, DZ), f32),
            pltpu.VMEM((B, DZ), f32),
            pltpu.VMEM((YP, B), f32),
            pltpu.VMEM((YP, B), f32),
            pltpu.VMEM((B, YP), f32),
            pltpu.VMEM((1, B), f32),
        ],
        compiler_params=pltpu.CompilerParams(
            dimension_semantics=("arbitrary", "arbitrary")),
    )(z, wqT, bq_eff, wkT, bk2, wg2p, wob, phi2, b2, eye)

    return outT[0:2, :].T


# speedup vs baseline: 2.7938x; 1.0700x over previous
import jax
import jax.numpy as jnp
from jax.experimental import pallas as pl
from jax.experimental.pallas import tpu as pltpu

B = 2048
T = 20
D_IN = 128
H = 128
D_DEMO = 16
DZ = H + D_DEMO
HEADS = 4
D_K = DZ // HEADS
GCN_DIM = 128
RB = 256
NRB = B // RB
YP = 8
XH = D_IN + H


def _nt(a, b):
    return jax.lax.dot_general(a, b, (((1,), (1,)), ((), ())),
                               preferred_element_type=jnp.float32)


def _body(x_hbm, xdemo_ref, sl_ref, wih_ref, whh_ref, bih_ref, bhh_ref,
          h0_ref, wq_ref, bq_ref, wk_ref, bk_ref, wow_ref, wob_ref,
          phi_ref, wg_ref, bg_ref, wpre_ref, bpre_ref, out_ref,
          h_s, xh_s, xbuf, sem, z_s, wcat_s, bcat_s, scale_s, eye_s,
          b2_s, s_s, q_s, k_s, y2t_s, y2c_s, ydt_s, st_s, dinvrow_s):
    s = pl.program_id(0)

    @pl.when(s == 0)
    def _():
        wcat_s[0:2 * H, 0:D_IN] = wih_ref[0:2 * H, :]
        wcat_s[0:2 * H, D_IN:XH] = whh_ref[0:2 * H, :]
        wcat_s[2 * H:3 * H, 0:D_IN] = wih_ref[2 * H:3 * H, :]
        wcat_s[2 * H:3 * H, D_IN:XH] = jnp.zeros((H, H), jnp.float32)
        wcat_s[3 * H:4 * H, 0:D_IN] = jnp.zeros((H, D_IN), jnp.float32)
        wcat_s[3 * H:4 * H, D_IN:XH] = whh_ref[2 * H:3 * H, :]
        bcat_s[:, 0:2 * H] = bih_ref[:, 0:2 * H] + bhh_ref[:, 0:2 * H]
        bcat_s[:, 2 * H:3 * H] = bih_ref[:, 2 * H:3 * H]
        bcat_s[:, 3 * H:4 * H] = bhh_ref[:, 2 * H:3 * H]
        for hh in range(HEADS):
            scale_s[:, hh * D_K:(hh + 1) * D_K] = jnp.broadcast_to(
                wow_ref[:, hh:hh + 1], (1, D_K)) / jnp.sqrt(jnp.float32(D_K))
        r_iota = jax.lax.broadcasted_iota(jnp.int32, (RB, RB), 0)
        c_iota = jax.lax.broadcasted_iota(jnp.int32, (RB, RB), 1)
        eye_s[...] = (r_iota == c_iota).astype(jnp.float32)
        b2_s[...] = _nt(bg_ref[...], wpre_ref[...]) + bpre_ref[...]
        h_s[...] = jnp.broadcast_to(h0_ref[...], (B, H))
        pltpu.make_async_copy(x_hbm.at[:, 0, :], xbuf.at[0], sem.at[0]).start()

    @pl.when(s < T)
    def _():
        slot = jax.lax.rem(s, 2)
        nslot = jax.lax.rem(s + 1, 2)

        @pl.when(s + 1 < T)
        def _():
            pltpu.make_async_copy(x_hbm.at[:, s + 1, :], xbuf.at[nslot],
                                  sem.at[nslot]).start()

        pltpu.make_async_copy(x_hbm.at[:, s, :], xbuf.at[slot],
                              sem.at[slot]).wait()
        h = h_s[...]
        xh_s[:, 0:D_IN] = xbuf[slot]
        xh_s[:, D_IN:XH] = h
        g = _nt(xh_s[...], wcat_s[...]) + bcat_s[...]
        rz = 0.5 * (jnp.tanh(g[:, 0:2 * H] * 0.5) + 1.0)
        r = rz[:, 0:H]
        zg = rz[:, H:2 * H]
        n = jnp.tanh(g[:, 2 * H:3 * H] + r * g[:, 3 * H:4 * H])
        h_new = (1.0 - zg) * n + zg * h
        h_s[...] = h_new
        idx = jnp.clip(sl_ref[...] - 1, 0, T - 1)
        z_s[:, 0:H] = jnp.where(idx == s, h_new, z_s[:, 0:H])

        @pl.when(s == T - 1)
        def _():
            z_s[:, H:DZ] = xdemo_ref[...]

    @pl.when(s == T)
    def _():
        z = z_s[...]
        q_s[...] = (_nt(z, wq_ref[...]) + bq_ref[...]) * scale_s[...]
        k_s[...] = _nt(z, wk_ref[...]) + bk_ref[...]
        wg2v = jax.lax.dot_general(wpre_ref[...], wg_ref[...],
                                   (((1,), (0,)), ((), ())),
                                   preferred_element_type=jnp.float32)
        y2t_s[0:2, :] = _nt(wg2v, z)
        y2t_s[2:YP, :] = jnp.zeros((YP - 2, B), jnp.float32)
        y2c_s[...] = jnp.concatenate(
            [_nt(z, wg2v), jnp.zeros((B, YP - 2), jnp.float32)], axis=1)

    @pl.when(jnp.logical_and(s >= T, s < T + NRB))
    def _():
        i = s - T
        q_i = q_s[pl.ds(i * RB, RB), :]
        sc = _nt(q_i, k_s[...]) + wob_ref[...]
        s_s[pl.ds(i * RB, RB), :] = sc
        m = jnp.max(sc, axis=1, keepdims=True)
        e = jnp.exp(sc - m)
        den = jnp.sum(e, axis=1, keepdims=True)
        thr = m + jnp.log(phi_ref[...] * den)
        deg = jnp.sum((sc >= thr).astype(jnp.float32), axis=1,
                      keepdims=True) + 1.0
        dinv = jax.lax.rsqrt(deg)
        st_s[pl.ds(i * RB, RB), :] = jnp.concatenate(
            [thr, dinv, jnp.zeros((RB, YP - 2), jnp.float32)], axis=1)
        dinvrow_s[:, pl.ds(i * RB, RB)] = jax.lax.dot_general(
            dinv, eye_s[...], (((0,), (0,)), ((), ())),
            preferred_element_type=jnp.float32)

    @pl.when(s >= T + NRB)
    def _():
        i = s - T - NRB

        @pl.when(i == 0)
        def _():
            ydt_s[...] = y2t_s[...] * dinvrow_s[...]

        sc = s_s[pl.ds(i * RB, RB), :]
        st = st_s[pl.ds(i * RB, RB), :]
        thr = st[:, 0:1]
        dinv_i = st[:, 1:2]
        maskf = (sc >= thr).astype(jnp.float32)
        accT = _nt(ydt_s[...], maskf)
        accF = _nt(eye_s[...], accT)
        outb = dinv_i * accF + (dinv_i * dinv_i) * y2c_s[pl.ds(i * RB, RB), :]
        out_ref[...] = outb[:, 0:2] + b2_s[...]


def kernel(x, x_demo, sorted_length, W_ih, W_hh, b_ih, b_hh, h0, Wq, bq,
           Wk, bk, Wo_w, Wo_b, phi, Wg, bg, W_pre, b_pre):
    f32 = jnp.float32
    sl2d = sorted_length.astype(jnp.int32).reshape(B, 1)
    const = lambda bs: pl.BlockSpec(bs, lambda s: tuple(0 for _ in bs))

    return pl.pallas_call(
        _body,
        grid=(T + 2 * NRB,),
        in_specs=[
            pl.BlockSpec(memory_space=pl.ANY),
            const((B, D_DEMO)),
            const((B, 1)),
            const((3 * H, D_IN)),
            const((3 * H, H)),
            const((1, 3 * H)),
            const((1, 3 * H)),
            const((1, H)),
            const((DZ, DZ)),
            const((1, DZ)),
            const((DZ, DZ)),
            const((1, DZ)),
            const((1, HEADS)),
            const((1, 1)),
            const((1, 1)),
            const((GCN_DIM, DZ)),
            const((1, GCN_DIM)),
            const((2, GCN_DIM)),
            const((1, 2)),
        ],
        out_specs=pl.BlockSpec(
            (RB, 2), lambda s: (jnp.clip(s - T - NRB, 0, NRB - 1), 0)),
        out_shape=jax.ShapeDtypeStruct((B, 2), f32),
        scratch_shapes=[
            pltpu.VMEM((B, H), f32),
            pltpu.VMEM((B, XH), f32),
            pltpu.VMEM((2, B, D_IN), f32),
            pltpu.SemaphoreType.DMA((2,)),
            pltpu.VMEM((B, DZ), f32),
            pltpu.VMEM((4 * H, XH), f32),
            pltpu.VMEM((1, 4 * H), f32),
            pltpu.VMEM((1, DZ), f32),
            pltpu.VMEM((RB, RB), f32),
            pltpu.VMEM((1, 2), f32),
            pltpu.VMEM((B, B), f32),
            pltpu.VMEM((B, DZ), f32),
            pltpu.VMEM((B, DZ), f32),
            pltpu.VMEM((YP, B), f32),
            pltpu.VMEM((B, YP), f32),
            pltpu.VMEM((YP, B), f32),
            pltpu.VMEM((B, YP), f32),
            pltpu.VMEM((1, B), f32),
        ],
        compiler_params=pltpu.CompilerParams(
            dimension_semantics=("arbitrary",)),
    )(x, x_demo, sl2d, W_ih, W_hh, b_ih.reshape(1, 3 * H),
      b_hh.reshape(1, 3 * H), h0.reshape(1, H), Wq, bq.reshape(1, DZ),
      Wk, bk.reshape(1, DZ), Wo_w, Wo_b.reshape(1, 1),
      jnp.asarray(phi, f32).reshape(1, 1), Wg, bg.reshape(1, GCN_DIM),
      W_pre, b_pre.reshape(1, 2))


# speedup vs baseline: 4.0897x; 1.4639x over previous
import jax
import jax.numpy as jnp
from jax.experimental import pallas as pl
from jax.experimental.pallas import tpu as pltpu

B = 2048
T = 20
D_IN = 128
H = 128
D_DEMO = 16
DZ = H + D_DEMO
HEADS = 4
D_K = DZ // HEADS
GCN_DIM = 128
RB = 256
NRB = B // RB
YP = 8
XH = D_IN + H


def _nt(a, b):
    return jax.lax.dot_general(a, b, (((1,), (1,)), ((), ())),
                               preferred_element_type=jnp.float32)


def _body(x_hbm, xdemo_ref, sl_ref, wih_ref, whh_ref, bih_ref, bhh_ref,
          h0_ref, wq_ref, bq_ref, wk_ref, bk_ref, wow_ref, wob_ref,
          phi_ref, wg_ref, bg_ref, wpre_ref, bpre_ref, out_ref,
          h_s, xh_s, xbuf, sem, z_s, wcat_s, bcat_s, scale_s, eye_s,
          b2_s, s_s, q_s, k_s, y2t_s, y2c_s, ydt_s, st_s, dinvrow_s):
    s = pl.program_id(0)

    @pl.when(s == 0)
    def _():
        wcat_s[0:2 * H, 0:D_IN] = wih_ref[0:2 * H, :]
        wcat_s[0:2 * H, D_IN:XH] = whh_ref[0:2 * H, :]
        wcat_s[2 * H:3 * H, 0:D_IN] = wih_ref[2 * H:3 * H, :]
        wcat_s[2 * H:3 * H, D_IN:XH] = jnp.zeros((H, H), jnp.float32)
        wcat_s[3 * H:4 * H, 0:D_IN] = jnp.zeros((H, D_IN), jnp.float32)
        wcat_s[3 * H:4 * H, D_IN:XH] = whh_ref[2 * H:3 * H, :]
        bcat_s[:, 0:2 * H] = bih_ref[:, 0:2 * H] + bhh_ref[:, 0:2 * H]
        bcat_s[:, 2 * H:3 * H] = bih_ref[:, 2 * H:3 * H]
        bcat_s[:, 3 * H:4 * H] = bhh_ref[:, 2 * H:3 * H]
        for hh in range(HEADS):
            scale_s[:, hh * D_K:(hh + 1) * D_K] = jnp.broadcast_to(
                wow_ref[:, hh:hh + 1], (1, D_K)) / jnp.sqrt(jnp.float32(D_K))
        r_iota = jax.lax.broadcasted_iota(jnp.int32, (RB, RB), 0)
        c_iota = jax.lax.broadcasted_iota(jnp.int32, (RB, RB), 1)
        eye_s[...] = (r_iota == c_iota).astype(jnp.float32)
        b2_s[...] = _nt(bg_ref[...], wpre_ref[...]) + bpre_ref[...]
        h_s[...] = jnp.broadcast_to(h0_ref[...], (B, H))
        pltpu.make_async_copy(x_hbm.at[0], xbuf.at[0], sem.at[0]).start()

    @pl.when(s < T)
    def _():
        slot = jax.lax.rem(s, 2)
        nslot = jax.lax.rem(s + 1, 2)

        @pl.when(s + 1 < T)
        def _():
            pltpu.make_async_copy(x_hbm.at[s + 1], xbuf.at[nslot],
                                  sem.at[nslot]).start()

        pltpu.make_async_copy(x_hbm.at[s], xbuf.at[slot],
                              sem.at[slot]).wait()
        h = h_s[...]
        xh_s[:, 0:D_IN] = xbuf[slot]
        xh_s[:, D_IN:XH] = h
        g = _nt(xh_s[...], wcat_s[...]) + bcat_s[...]
        rz = 0.5 * (jnp.tanh(g[:, 0:2 * H] * 0.5) + 1.0)
        r = rz[:, 0:H]
        zg = rz[:, H:2 * H]
        n = jnp.tanh(g[:, 2 * H:3 * H] + r * g[:, 3 * H:4 * H])
        h_new = (1.0 - zg) * n + zg * h
        h_s[...] = h_new
        idx = jnp.clip(sl_ref[...] - 1, 0, T - 1)
        z_s[:, 0:H] = jnp.where(idx == s, h_new, z_s[:, 0:H])

        @pl.when(s == T - 1)
        def _():
            z_s[:, H:DZ] = xdemo_ref[...]

    @pl.when(s == T)
    def _():
        z = z_s[...]
        q_s[...] = (_nt(z, wq_ref[...]) + bq_ref[...]) * scale_s[...]
        k_s[...] = _nt(z, wk_ref[...]) + bk_ref[...]
        wg2v = jax.lax.dot_general(wpre_ref[...], wg_ref[...],
                                   (((1,), (0,)), ((), ())),
                                   preferred_element_type=jnp.float32)
        y2t_s[0:2, :] = _nt(wg2v, z)
        y2t_s[2:YP, :] = jnp.zeros((YP - 2, B), jnp.float32)
        y2c_s[...] = jnp.concatenate(
            [_nt(z, wg2v), jnp.zeros((B, YP - 2), jnp.float32)], axis=1)

    @pl.when(jnp.logical_and(s >= T, s < T + NRB))
    def _():
        i = s - T
        q_i = q_s[pl.ds(i * RB, RB), :]
        sc = _nt(q_i, k_s[...]) + wob_ref[...]
        s_s[pl.ds(i * RB, RB), :] = sc
        m = jnp.max(sc, axis=1, keepdims=True)
        e = jnp.exp(sc - m)
        den = jnp.sum(e, axis=1, keepdims=True)
        thr = m + jnp.log(phi_ref[...] * den)
        deg = jnp.sum((sc >= thr).astype(jnp.float32), axis=1,
                      keepdims=True) + 1.0
        dinv = jax.lax.rsqrt(deg)
        st_s[pl.ds(i * RB, RB), :] = jnp.concatenate(
            [thr, dinv, jnp.zeros((RB, YP - 2), jnp.float32)], axis=1)
        dinvrow_s[:, pl.ds(i * RB, RB)] = jax.lax.dot_general(
            dinv, eye_s[...], (((0,), (0,)), ((), ())),
            preferred_element_type=jnp.float32)

    @pl.when(s >= T + NRB)
    def _():
        i = s - T - NRB

        @pl.when(i == 0)
        def _():
            ydt_s[...] = y2t_s[...] * dinvrow_s[...]

        sc = s_s[pl.ds(i * RB, RB), :]
        st = st_s[pl.ds(i * RB, RB), :]
        thr = st[:, 0:1]
        dinv_i = st[:, 1:2]
        maskf = (sc >= thr).astype(jnp.float32)
        accT = _nt(ydt_s[...], maskf)
        accF = _nt(eye_s[...], accT)
        outb = dinv_i * accF + (dinv_i * dinv_i) * y2c_s[pl.ds(i * RB, RB), :]
        out_ref[...] = outb[:, 0:2] + b2_s[...]


def kernel(x, x_demo, sorted_length, W_ih, W_hh, b_ih, b_hh, h0, Wq, bq,
           Wk, bk, Wo_w, Wo_b, phi, Wg, bg, W_pre, b_pre):
    f32 = jnp.float32
    xt = jnp.swapaxes(x, 0, 1)
    sl2d = sorted_length.astype(jnp.int32).reshape(B, 1)
    const = lambda bs: pl.BlockSpec(bs, lambda s: tuple(0 for _ in bs))

    return pl.pallas_call(
        _body,
        grid=(T + 2 * NRB,),
        in_specs=[
            pl.BlockSpec(memory_space=pl.ANY),
            const((B, D_DEMO)),
            const((B, 1)),
            const((3 * H, D_IN)),
            const((3 * H, H)),
            const((1, 3 * H)),
            const((1, 3 * H)),
            const((1, H)),
            const((DZ, DZ)),
            const((1, DZ)),
            const((DZ, DZ)),
            const((1, DZ)),
            const((1, HEADS)),
            const((1, 1)),
            const((1, 1)),
            const((GCN_DIM, DZ)),
            const((1, GCN_DIM)),
            const((2, GCN_DIM)),
            const((1, 2)),
        ],
        out_specs=pl.BlockSpec(
            (RB, 2), lambda s: (jnp.clip(s - T - NRB, 0, NRB - 1), 0)),
        out_shape=jax.ShapeDtypeStruct((B, 2), f32),
        scratch_shapes=[
            pltpu.VMEM((B, H), f32),
            pltpu.VMEM((B, XH), f32),
            pltpu.VMEM((2, B, D_IN), f32),
            pltpu.SemaphoreType.DMA((2,)),
            pltpu.VMEM((B, DZ), f32),
            pltpu.VMEM((4 * H, XH), f32),
            pltpu.VMEM((1, 4 * H), f32),
            pltpu.VMEM((1, DZ), f32),
            pltpu.VMEM((RB, RB), f32),
            pltpu.VMEM((1, 2), f32),
            pltpu.VMEM((B, B), f32),
            pltpu.VMEM((B, DZ), f32),
            pltpu.VMEM((B, DZ), f32),
            pltpu.VMEM((YP, B), f32),
            pltpu.VMEM((B, YP), f32),
            pltpu.VMEM((YP, B), f32),
            pltpu.VMEM((B, YP), f32),
            pltpu.VMEM((1, B), f32),
        ],
        compiler_params=pltpu.CompilerParams(
            dimension_semantics=("arbitrary",)),
    )(xt, x_demo, sl2d, W_ih, W_hh, b_ih.reshape(1, 3 * H),
      b_hh.reshape(1, 3 * H), h0.reshape(1, H), Wq, bq.reshape(1, DZ),
      Wk, bk.reshape(1, DZ), Wo_w, Wo_b.reshape(1, 1),
      jnp.asarray(phi, f32).reshape(1, 1), Wg, bg.reshape(1, GCN_DIM),
      W_pre, b_pre.reshape(1, 2))


# speedup vs baseline: 4.9575x; 1.2122x over previous
import jax
import jax.numpy as jnp
from jax.experimental import pallas as pl
from jax.experimental.pallas import tpu as pltpu

B = 2048
T = 20
D_IN = 128
H = 128
D_DEMO = 16
DZ = H + D_DEMO
HEADS = 4
D_K = DZ // HEADS
GCN_DIM = 128
RB = 256
NRB = B // RB
YP = 8
XH = D_IN + H


def _nt(a, b):
    return jax.lax.dot_general(a, b, (((1,), (1,)), ((), ())),
                               preferred_element_type=jnp.float32)


def _body(x_hbm, xdemoT_ref, idx_ref, wih_ref, whh_ref, bih_ref, bhh_ref,
          h0_ref, wq_ref, bq_ref, wk_ref, bk_ref, wow_ref, wob_ref,
          phi_ref, wgT_ref, bg_ref, wpre_ref, bpre_ref, out_ref,
          h_s, xh_s, xbuf, sem, z_s, wcat_s, bcat_s, scale_s, eye_s,
          eye16_s, b2_s, s_s, q_s, k_s, y2t_s, ydt_s, st_s, dinvrow_s):
    s = pl.program_id(0)

    @pl.when(s == 0)
    def _():
        wcat_s[0:2 * H, 0:D_IN] = wih_ref[0:2 * H, :]
        wcat_s[0:2 * H, D_IN:XH] = whh_ref[0:2 * H, :]
        wcat_s[2 * H:3 * H, 0:D_IN] = wih_ref[2 * H:3 * H, :]
        wcat_s[2 * H:3 * H, D_IN:XH] = jnp.zeros((H, H), jnp.float32)
        wcat_s[3 * H:4 * H, 0:D_IN] = jnp.zeros((H, D_IN), jnp.float32)
        wcat_s[3 * H:4 * H, D_IN:XH] = whh_ref[2 * H:3 * H, :]
        bcat_s[:, 0:2 * H] = bih_ref[:, 0:2 * H] + bhh_ref[:, 0:2 * H]
        bcat_s[:, 2 * H:3 * H] = bih_ref[:, 2 * H:3 * H]
        bcat_s[:, 3 * H:4 * H] = bhh_ref[:, 2 * H:3 * H]
        for hh in range(HEADS):
            scale_s[:, hh * D_K:(hh + 1) * D_K] = jnp.broadcast_to(
                wow_ref[:, hh:hh + 1], (1, D_K)) / jnp.sqrt(jnp.float32(D_K))
        r_iota = jax.lax.broadcasted_iota(jnp.int32, (RB, RB), 0)
        c_iota = jax.lax.broadcasted_iota(jnp.int32, (RB, RB), 1)
        eye_s[...] = (r_iota == c_iota).astype(jnp.float32)
        r16 = jax.lax.broadcasted_iota(jnp.int32, (D_DEMO, D_DEMO), 0)
        c16 = jax.lax.broadcasted_iota(jnp.int32, (D_DEMO, D_DEMO), 1)
        eye16_s[...] = (r16 == c16).astype(jnp.float32)
        b2_s[...] = jnp.broadcast_to(_nt(wpre_ref[...], bg_ref[...]),
                                     (2, RB)) + bpre_ref[...]
        h_s[...] = jnp.broadcast_to(h0_ref[...], (B, H))
        pltpu.make_async_copy(x_hbm.at[0], xbuf.at[0], sem.at[0]).start()

    @pl.when(s < T)
    def _():
        slot = jax.lax.rem(s, 2)
        nslot = jax.lax.rem(s + 1, 2)

        @pl.when(s + 1 < T)
        def _():
            pltpu.make_async_copy(x_hbm.at[s + 1], xbuf.at[nslot],
                                  sem.at[nslot]).start()

        pltpu.make_async_copy(x_hbm.at[s], xbuf.at[slot],
                              sem.at[slot]).wait()
        h = h_s[...]
        xh_s[:, 0:D_IN] = xbuf[slot]
        xh_s[:, D_IN:XH] = h
        g = _nt(xh_s[...], wcat_s[...]) + bcat_s[...]
        rz = 0.5 * (jnp.tanh(g[:, 0:2 * H] * 0.5) + 1.0)
        r = rz[:, 0:H]
        zg = rz[:, H:2 * H]
        n = jnp.tanh(g[:, 2 * H:3 * H] + r * g[:, 3 * H:4 * H])
        h_new = (1.0 - zg) * n + zg * h
        h_s[...] = h_new
        z_s[:, 0:H] = jnp.where(idx_ref[...] == s.astype(jnp.float32),
                                h_new, z_s[:, 0:H])

        @pl.when(s == T - 1)
        def _():
            z_s[:, H:DZ] = jax.lax.dot_general(
                xdemoT_ref[...], eye16_s[...], (((0,), (0,)), ((), ())),
                preferred_element_type=jnp.float32)

    @pl.when(s == T)
    def _():
        z = z_s[...]
        q_s[...] = (_nt(z, wq_ref[...]) + bq_ref[...]) * scale_s[...]
        k_s[...] = _nt(z, wk_ref[...]) + bk_ref[...]
        wg2v = _nt(wpre_ref[...], wgT_ref[...])
        y2t_s[0:2, :] = _nt(wg2v, z)
        y2t_s[2:YP, :] = jnp.zeros((YP - 2, B), jnp.float32)

    @pl.when(jnp.logical_and(s >= T, s < T + NRB))
    def _():
        i = s - T
        q_i = q_s[pl.ds(i * RB, RB), :]
        sc = _nt(q_i, k_s[...]) + wob_ref[...]
        s_s[pl.ds(i * RB, RB), :] = sc
        m = jnp.max(sc, axis=1, keepdims=True)
        e = jnp.exp(sc - m)
        den = jnp.sum(e, axis=1, keepdims=True)
        thr = m + jnp.log(phi_ref[...] * den)
        deg = jnp.sum((sc >= thr).astype(jnp.float32), axis=1,
                      keepdims=True) + 1.0
        dinv = jax.lax.rsqrt(deg)
        st_s[pl.ds(i * RB, RB), :] = jnp.concatenate(
            [thr, dinv, jnp.zeros((RB, YP - 2), jnp.float32)], axis=1)
        dinvrow_s[:, pl.ds(i * RB, RB)] = jax.lax.dot_general(
            dinv, eye_s[...], (((0,), (0,)), ((), ())),
            preferred_element_type=jnp.float32)

    @pl.when(s >= T + NRB)
    def _():
        i = s - T - NRB

        @pl.when(i == 0)
        def _():
            ydt_s[...] = y2t_s[...] * dinvrow_s[...]

        sc = s_s[pl.ds(i * RB, RB), :]
        thr = st_s[pl.ds(i * RB, RB), 0:1]
        maskf = (sc >= thr).astype(jnp.float32)
        accT = _nt(ydt_s[...], maskf)
        dr_i = dinvrow_s[:, pl.ds(i * RB, RB)]
        y2t_i = y2t_s[0:2, pl.ds(i * RB, RB)]
        out_ref[...] = dr_i * accT[0:2, :] + (dr_i * dr_i) * y2t_i + b2_s[...]


def kernel(x, x_demo, sorted_length, W_ih, W_hh, b_ih, b_hh, h0, Wq, bq,
           Wk, bk, Wo_w, Wo_b, phi, Wg, bg, W_pre, b_pre):
    f32 = jnp.float32
    xt = jnp.swapaxes(x, 0, 1)
    xdemoT = x_demo.T
    wgT = Wg.T
    idxb = jnp.broadcast_to(
        jnp.clip(sorted_length.astype(f32) - 1, 0, T - 1)[:, None], (B, H))
    const = lambda bs: pl.BlockSpec(bs, lambda s: tuple(0 for _ in bs))

    outT = pl.pallas_call(
        _body,
        grid=(T + 2 * NRB,),
        in_specs=[
            pl.BlockSpec(memory_space=pl.ANY),
            const((D_DEMO, B)),
            const((B, H)),
            const((3 * H, D_IN)),
            const((3 * H, H)),
            const((1, 3 * H)),
            const((1, 3 * H)),
            const((1, H)),
            const((DZ, DZ)),
            const((1, DZ)),
            const((DZ, DZ)),
            const((1, DZ)),
            const((1, HEADS)),
            const((1, 1)),
            const((1, 1)),
            const((DZ, GCN_DIM)),
            const((1, GCN_DIM)),
            const((2, GCN_DIM)),
            const((2, RB)),
        ],
        out_specs=pl.BlockSpec(
            (2, RB), lambda s: (0, jnp.clip(s - T - NRB, 0, NRB - 1))),
        out_shape=jax.ShapeDtypeStruct((2, B), f32),
        scratch_shapes=[
            pltpu.VMEM((B, H), f32),
            pltpu.VMEM((B, XH), f32),
            pltpu.VMEM((2, B, D_IN), f32),
            pltpu.SemaphoreType.DMA((2,)),
            pltpu.VMEM((B, DZ), f32),
            pltpu.VMEM((4 * H, XH), f32),
            pltpu.VMEM((1, 4 * H), f32),
            pltpu.VMEM((1, DZ), f32),
            pltpu.VMEM((RB, RB), f32),
            pltpu.VMEM((D_DEMO, D_DEMO), f32),
            pltpu.VMEM((2, RB), f32),
            pltpu.VMEM((B, B), f32),
            pltpu.VMEM((B, DZ), f32),
            pltpu.VMEM((B, DZ), f32),
            pltpu.VMEM((YP, B), f32),
            pltpu.VMEM((YP, B), f32),
            pltpu.VMEM((B, YP), f32),
            pltpu.VMEM((1, B), f32),
        ],
        compiler_params=pltpu.CompilerParams(
            dimension_semantics=("arbitrary",)),
    )(xt, xdemoT, idxb, W_ih, W_hh, b_ih.reshape(1, 3 * H),
      b_hh.reshape(1, 3 * H), h0.reshape(1, H), Wq, bq.reshape(1, DZ),
      Wk, bk.reshape(1, DZ), Wo_w, Wo_b.reshape(1, 1),
      jnp.asarray(phi, f32).reshape(1, 1), wgT, bg.reshape(1, GCN_DIM),
      W_pre, jnp.broadcast_to(b_pre.reshape(2, 1), (2, RB)))
    return outT.T
